# Initial kernel scaffold; baseline (speedup 1.0000x reference)
import jax
import jax.numpy as jnp
from jax.experimental import pallas as pl


def kernel(x, wave_w, wave_b, kernel_w, kernel_b, out_w):
    raise NotImplementedError("write your pallas kernel here")


# speedup vs baseline: 2.4279x; 2.4279x over previous
import functools

import jax
import jax.numpy as jnp
from jax.experimental import pallas as pl
from jax.experimental.pallas import tpu as pltpu

L = 2048
C = 1024
H = 16
K = 64
HALF_S = 16
S = 2 * HALF_S + 1
MAX_FREQ = 16.0
MIN_FREQ = 1.0

LBLK_A = 256
LBLK_B = 64
LBLK_C = 256


def _silu(v):
    return v * jax.nn.sigmoid(v)


def _proj_kernel(x_ref, ww_ref, wb_ref, kw_ref, kb_ref, pos_ref, w_ref):
    i = pl.program_id(0)
    xb = x_ref[...]
    wave = _silu(
        jax.lax.dot_general(xb, ww_ref[...], (((1,), (1,)), ((), ())),
                            precision=jax.lax.Precision.DEFAULT)
        + wb_ref[...])
    freq = jax.nn.sigmoid(wave[:, :H]) * (MAX_FREQ - MIN_FREQ) + MIN_FREQ
    phase = jnp.tanh(wave[:, H:]) * MAX_FREQ
    freq_avg = jnp.mean(freq, axis=1, keepdims=True)
    phase_avg = jnp.mean(phase, axis=1, keepdims=True)
    s_off = (jax.lax.broadcasted_iota(jnp.int32, (1, S), 1)
             .astype(jnp.float32) - HALF_S)
    base = ((i * LBLK_A).astype(jnp.float32)
            + jax.lax.broadcasted_iota(jnp.int32, (LBLK_A, 1), 0)
            .astype(jnp.float32))
    offsets = phase_avg + s_off * freq_avg
    posf = jnp.clip(jnp.round(base + offsets), 0, L - 1)
    pos_ref[...] = posf.astype(jnp.int32)
    kb = _silu(
        jax.lax.dot_general(xb, kw_ref[...], (((1,), (1,)), ((), ())),
                            precision=jax.lax.Precision.DEFAULT)
        + kb_ref[...])
    w_ref[...] = kb


def _gather_kernel(x_ref, pos_ref, w_ref, out_ref):
    def body(t, _):
        wt = w_ref[t]
        acc = jnp.zeros((H, K), dtype=jnp.float32)
        for s in range(S):
            p = pos_ref[t, s]
            row = x_ref[p]
            acc = acc + wt[:, s:s + 1] * row
        out_ref[t] = acc
        return 0

    jax.lax.fori_loop(0, LBLK_B, body, 0)


def _out_kernel(h_ref, ow_ref, o_ref):
    o_ref[...] = _silu(
        jax.lax.dot_general(h_ref[...], ow_ref[...], (((1,), (1,)), ((), ())),
                            precision=jax.lax.Precision.DEFAULT))


@functools.partial(jax.jit, static_argnames=("interpret",))
def kernel(x, wave_w, wave_b, kernel_w, kernel_b, out_w, interpret=False):
    b, l, c = x.shape
    x2 = x.reshape(l, c)
    kw_r = kernel_w.reshape(H, K, c)[:, :S].reshape(H * S, c)
    kb_r = kernel_b.reshape(H, K)[:, :S].reshape(H * S)

    pos, w = pl.pallas_call(
        _proj_kernel,
        grid=(l // LBLK_A,),
        in_specs=[
            pl.BlockSpec((LBLK_A, c), lambda i: (i, 0)),
            pl.BlockSpec((2 * H, c), lambda i: (0, 0)),
            pl.BlockSpec((1, 2 * H), lambda i: (0, 0)),
            pl.BlockSpec((H * S, c), lambda i: (0, 0)),
            pl.BlockSpec((1, H * S), lambda i: (0, 0)),
        ],
        out_specs=[
            pl.BlockSpec((LBLK_A, S), lambda i: (i, 0)),
            pl.BlockSpec((LBLK_A, H * S), lambda i: (i, 0)),
        ],
        out_shape=[
            jax.ShapeDtypeStruct((l, S), jnp.int32),
            jax.ShapeDtypeStruct((l, H * S), jnp.float32),
        ],
        interpret=interpret,
    )(x2, wave_w, wave_b.reshape(1, 2 * H), kw_r, kb_r.reshape(1, H * S))

    w3 = w.reshape(l, H, S)
    x3 = x2.reshape(l, H, K)

    out_h = pl.pallas_call(
        _gather_kernel,
        grid=(l // LBLK_B,),
        in_specs=[
            pl.BlockSpec((l, H, K), lambda i: (0, 0, 0)),
            pl.BlockSpec((LBLK_B, S), lambda i: (i, 0),
                         memory_space=pltpu.SMEM),
            pl.BlockSpec((LBLK_B, H, S), lambda i: (i, 0, 0)),
        ],
        out_specs=pl.BlockSpec((LBLK_B, H, K), lambda i: (i, 0, 0)),
        out_shape=jax.ShapeDtypeStruct((l, H, K), jnp.float32),
        interpret=interpret,
    )(x3, pos, w3)

    out = pl.pallas_call(
        _out_kernel,
        grid=(l // LBLK_C,),
        in_specs=[
            pl.BlockSpec((LBLK_C, c), lambda i: (i, 0)),
            pl.BlockSpec((c, c), lambda i: (0, 0)),
        ],
        out_specs=pl.BlockSpec((LBLK_C, c), lambda i: (i, 0)),
        out_shape=jax.ShapeDtypeStruct((l, c), jnp.float32),
        interpret=interpret,
    )(out_h.reshape(l, c), out_w)

    return out.reshape(b, l, c)
